# baseline (speedup 1.0000x reference)
import jax
import jax.numpy as jnp
from jax.experimental import pallas as pl

_BM = 200


def _fused_gcn_kernel(c_ref, x_ref, wt_ref, b_ref, o_ref):
    prop = jnp.dot(c_ref[...], x_ref[...], preferred_element_type=jnp.float32)
    o_ref[...] = (
        jnp.dot(prop, wt_ref[...], preferred_element_type=jnp.float32)
        + b_ref[...]
    )


def kernel(x, edge_index, C, W, b):
    del edge_index
    n, d_in = x.shape
    d_out = W.shape[0]
    x = x.astype(jnp.float32)
    C = C.astype(jnp.float32)
    wt = W.astype(jnp.float32).T
    b2 = b.astype(jnp.float32).reshape(1, d_out)

    z = lambda i: i * 0
    return pl.pallas_call(
        _fused_gcn_kernel,
        grid=(n // _BM,),
        in_specs=[
            pl.BlockSpec((_BM, n), lambda i: (i, z(i))),
            pl.BlockSpec((n, d_in), lambda i: (z(i), z(i))),
            pl.BlockSpec((d_in, d_out), lambda i: (z(i), z(i))),
            pl.BlockSpec((1, d_out), lambda i: (z(i), z(i))),
        ],
        out_specs=pl.BlockSpec((_BM, d_out), lambda i: (i, z(i))),
        out_shape=jax.ShapeDtypeStruct((n, d_out), jnp.float32),
    )(C, x, wt, b2)


# speedup vs baseline: 1.0200x; 1.0200x over previous
import jax
import jax.numpy as jnp
from jax.experimental import pallas as pl
from jax.experimental.pallas import tpu as pltpu

_BM = 400


def _fused_gcn_kernel(c_ref, x_ref, wt_ref, b_ref, o_ref):
    prop = jnp.dot(c_ref[...], x_ref[...], preferred_element_type=jnp.float32)
    o_ref[...] = (
        jnp.dot(prop, wt_ref[...], preferred_element_type=jnp.float32)
        + b_ref[...]
    )


def kernel(x, edge_index, C, W, b):
    del edge_index
    n, d_in = x.shape
    d_out = W.shape[0]
    x = x.astype(jnp.float32)
    C = C.astype(jnp.float32)
    wt = W.astype(jnp.float32).T
    b2 = b.astype(jnp.float32).reshape(1, d_out)

    z = lambda i: i * 0
    return pl.pallas_call(
        _fused_gcn_kernel,
        grid=(n // _BM,),
        in_specs=[
            pl.BlockSpec((_BM, n), lambda i: (i, z(i))),
            pl.BlockSpec((n, d_in), lambda i: (z(i), z(i))),
            pl.BlockSpec((d_in, d_out), lambda i: (z(i), z(i))),
            pl.BlockSpec((1, d_out), lambda i: (z(i), z(i))),
        ],
        out_specs=pl.BlockSpec((_BM, d_out), lambda i: (i, z(i))),
        out_shape=jax.ShapeDtypeStruct((n, d_out), jnp.float32),
        compiler_params=pltpu.CompilerParams(
            vmem_limit_bytes=112 * 1024 * 1024,
        ),
    )(C, x, wt, b2)


# speedup vs baseline: 1.0382x; 1.0178x over previous
import jax
import jax.numpy as jnp
from jax.experimental import pallas as pl
from jax.experimental.pallas import tpu as pltpu

_BM = 400


def _fused_gcn_kernel(c_ref, x_ref, w_ref, b_ref, o_ref):
    prop = jnp.dot(c_ref[...], x_ref[...], preferred_element_type=jnp.float32)
    lin = jax.lax.dot_general(
        prop, w_ref[...], (((1,), (1,)), ((), ())),
        preferred_element_type=jnp.float32,
    )
    o_ref[...] = lin + b_ref[...][None, :]


def kernel(x, edge_index, C, W, b):
    del edge_index
    n, d_in = x.shape
    d_out = W.shape[0]

    z = lambda i: i * 0
    return pl.pallas_call(
        _fused_gcn_kernel,
        grid=(n // _BM,),
        in_specs=[
            pl.BlockSpec((_BM, n), lambda i: (i, z(i))),
            pl.BlockSpec((n, d_in), lambda i: (z(i), z(i))),
            pl.BlockSpec((d_out, d_in), lambda i: (z(i), z(i))),
            pl.BlockSpec((d_out,), lambda i: (z(i),)),
        ],
        out_specs=pl.BlockSpec((_BM, d_out), lambda i: (i, z(i))),
        out_shape=jax.ShapeDtypeStruct((n, d_out), jnp.float32),
        compiler_params=pltpu.CompilerParams(
            vmem_limit_bytes=112 * 1024 * 1024,
        ),
    )(C, x, W, b)
